# baseline (speedup 1.0000x reference)
import functools

import jax
import jax.numpy as jnp
from jax import lax
from jax.experimental import pallas as pl
from jax.experimental.pallas import tpu as pltpu
from jax.experimental.pallas import tpu_sc as plsc

_THRESHOLD = 0.0
_C = 192
_H = 512
_W = 512
_BLK_H = 64
_NC = 2
_NS = 16
_NW = _NC * _NS
_ROWS_PER_W = _H // _NW
_LANES = 16
_CHUNKS_PER_ROW = _W // _LANES


def _sum_kernel(x_ref, sums_ref):
    sums_ref[...] = jnp.sum(x_ref[...], axis=0)


def _sc_bbox_kernel(sums_hbm, parts_hbm, rows_v, part_v):
    wid = lax.axis_index("s") * _NC + lax.axis_index("c")
    base = wid * _ROWS_PER_W
    pltpu.sync_copy(sums_hbm.at[pl.ds(base * _W, _ROWS_PER_W * _W)], rows_v)

    iota16 = lax.iota(jnp.int32, _LANES)

    def body(t, carry):
        rmin_v, rmax_v, cmin_v, cmax_v = carry
        v = rows_v[pl.ds(t * _LANES, _LANES)]
        m = v >= _THRESHOLD
        col_v = iota16 + (t % _CHUNKS_PER_ROW) * _LANES
        row_s = base + t // _CHUNKS_PER_ROW
        rmin_v = jnp.minimum(rmin_v, jnp.where(m, row_s, _H))
        rmax_v = jnp.maximum(rmax_v, jnp.where(m, row_s, -1))
        cmin_v = jnp.minimum(cmin_v, jnp.where(m, col_v, _W))
        cmax_v = jnp.maximum(cmax_v, jnp.where(m, col_v, -1))
        return rmin_v, rmax_v, cmin_v, cmax_v

    n_chunks = _ROWS_PER_W * _CHUNKS_PER_ROW
    init = (
        jnp.full((_LANES,), _H, jnp.int32),
        jnp.full((_LANES,), -1, jnp.int32),
        jnp.full((_LANES,), _W, jnp.int32),
        jnp.full((_LANES,), -1, jnp.int32),
    )
    rmin_v, rmax_v, cmin_v, cmax_v = lax.fori_loop(0, n_chunks, body, init)
    part_v[0, :] = rmin_v
    part_v[1, :] = rmax_v
    part_v[2, :] = cmin_v
    part_v[3, :] = cmax_v
    pltpu.sync_copy(part_v, parts_hbm.at[wid])


def _merge_kernel(parts_ref, out_ref, stats_ref):
    p = parts_ref[...]
    y1 = jnp.min(p[:, 0, :])
    y2 = jnp.max(p[:, 1, :]) + 1
    x1 = jnp.min(p[:, 2, :])
    x2 = jnp.max(p[:, 3, :]) + 1
    frac = jnp.where(
        y2 > 0,
        ((y2 - y1) * (x2 - x1)).astype(jnp.float32) / float(_H * _W),
        0.0,
    )
    stats_ref[...] = jnp.full((8, 128), frac, dtype=jnp.float32)
    out_ref[...] = jnp.zeros_like(out_ref)


@jax.jit
def kernel(x):
    xr = x.reshape(_C, _H, _W)
    sums = pl.pallas_call(
        _sum_kernel,
        grid=(_H // _BLK_H,),
        in_specs=[pl.BlockSpec((_C, _BLK_H, _W), lambda i: (0, i, 0))],
        out_specs=pl.BlockSpec((_BLK_H, _W), lambda i: (i, 0)),
        out_shape=jax.ShapeDtypeStruct((_H, _W), jnp.float32),
    )(xr)

    mesh = plsc.VectorSubcoreMesh(core_axis_name="c", subcore_axis_name="s")
    sc_fn = functools.partial(
        pl.kernel,
        mesh=mesh,
        out_type=jax.ShapeDtypeStruct((_NW, 4, _LANES), jnp.int32),
        scratch_types=[
            pltpu.VMEM((_ROWS_PER_W * _W,), jnp.float32),
            pltpu.VMEM((4, _LANES), jnp.int32),
        ],
    )(_sc_bbox_kernel)
    parts = sc_fn(sums.reshape(_H * _W))

    out, _stats = pl.pallas_call(
        _merge_kernel,
        out_shape=[
            jax.ShapeDtypeStruct((_H, _W), x.dtype),
            jax.ShapeDtypeStruct((8, 128), jnp.float32),
        ],
    )(parts)
    return out.reshape(1, 1, _H, _W)
